# Initial kernel scaffold; baseline (speedup 1.0000x reference)
import jax
import jax.numpy as jnp
from jax.experimental import pallas as pl


def kernel(point_feat, global_feat, ps1_w, ps1_b, m1c1_w, m1c1_b, m1c2_w, m1c2_b, m1sc_w, m1sc_b, m2c1_w, m2c1_b, m2c2_w, m2c2_b):
    raise NotImplementedError("write your pallas kernel here")


# speedup vs baseline: 2.1099x; 2.1099x over previous
import jax
import jax.numpy as jnp
from jax.experimental import pallas as pl
from jax.experimental.pallas import tpu as pltpu

_DIM = 1024
_N = 2048
_K = 64


def _fps_body(pts_ref, out_ref):
    iota = jax.lax.broadcasted_iota(jnp.int32, (_N, 1), 0)

    def step(i, carry):
        dist, far = carry
        c_row = pts_ref[0, pl.ds(far, 1), :]
        out_ref[0, pl.ds(i, 1), :] = c_row
        p = pts_ref[0]
        d = jnp.sum((p - c_row) ** 2, axis=1, keepdims=True)
        dist = jnp.minimum(dist, d)
        m = jnp.max(dist)
        nxt = jnp.min(jnp.where(dist == m, iota, _N))
        return dist, nxt

    dist0 = jnp.full((_N, 1), 1e10, dtype=jnp.float32)
    jax.lax.fori_loop(0, _K, step, (dist0, jnp.int32(0)))


def _ps1_body(g_ref, m_ref, out_ref):
    out_ref[...] = jnp.dot(g_ref[...], m_ref[...],
                           preferred_element_type=jnp.float32)


def _decoder_body(x1_ref, f_ref, g_ref,
                  a1_ref, b1_ref, c1_ref, sa_ref, sb_ref, sc_ref,
                  ps1b_ref, m1c1b_ref, m1scb_ref, m1c2w_ref, m1c2b_ref,
                  m2c1w_ref, m2c1b_ref, m2c2w_ref, m2c2b_ref, out_ref):
    dn = (((1,), (1,)), ((), ()))

    x1 = x1_ref[0] + ps1b_ref[...]
    f = f_ref[0]
    g = g_ref[0]

    def mm(w, x):
        return jnp.dot(w, x, preferred_element_type=jnp.float32)

    def mmt(w, x):
        return jax.lax.dot_general(w, x, dn, preferred_element_type=jnp.float32)

    h1 = mm(a1_ref[...], x1) + mmt(b1_ref[...], f) + mmt(c1_ref[...], g) \
        + m1c1b_ref[...]
    shortcut = mm(sa_ref[...], x1) + mmt(sb_ref[...], f) + mmt(sc_ref[...], g) \
        + m1scb_ref[...]
    h = mm(m1c2w_ref[...], jax.nn.relu(h1)) + m1c2b_ref[...] + shortcut
    r2 = jax.nn.relu(mm(m2c1w_ref[...], h) + m2c1b_ref[...])
    out_ref[0] = jax.lax.dot_general(
        r2, m2c2w_ref[...], (((0,), (1,)), ((), ())),
        preferred_element_type=jnp.float32) + m2c2b_ref[...]


def kernel(point_feat, global_feat, ps1_w, ps1_b, m1c1_w, m1c1_b, m1c2_w,
           m1c2_b, m1sc_w, m1sc_b, m2c1_w, m2c1_b, m2c2_w, m2c2_b):
    B = point_feat.shape[0]
    pts = jnp.transpose(point_feat, (0, 2, 1))

    cp = pltpu.CompilerParams(
        dimension_semantics=("arbitrary",),
        vmem_limit_bytes=100 * 1024 * 1024,
    )

    sampled = pl.pallas_call(
        _fps_body,
        grid=(B,),
        in_specs=[pl.BlockSpec((1, _N, _DIM), lambda b: (b, 0, 0))],
        out_specs=pl.BlockSpec((1, _K, _DIM), lambda b: (b, 0, 0)),
        out_shape=jax.ShapeDtypeStruct((B, _K, _DIM), jnp.float32),
        compiler_params=cp,
    )(pts)

    m = ps1_w.reshape(_DIM, 128 * _K)
    chunk = 1024
    x1_flat = pl.pallas_call(
        _ps1_body,
        grid=(128 * _K // chunk,),
        in_specs=[
            pl.BlockSpec((B, _DIM), lambda j: (0, 0)),
            pl.BlockSpec((_DIM, chunk), lambda j: (0, j)),
        ],
        out_specs=pl.BlockSpec((B, chunk), lambda j: (0, j)),
        out_shape=jax.ShapeDtypeStruct((B, 128 * _K), jnp.float32),
        compiler_params=cp,
    )(global_feat, m)
    x1 = x1_flat.reshape(B, 128, _K)

    a1, b1, c1 = m1c1_w[:, :128], m1c1_w[:, 128:128 + _DIM], m1c1_w[:, 128 + _DIM:]
    sa, sb, sc = m1sc_w[:, :128], m1sc_w[:, 128:128 + _DIM], m1sc_w[:, 128 + _DIM:]

    full = lambda shape: pl.BlockSpec(shape, lambda b: tuple(0 for _ in shape))
    out = pl.pallas_call(
        _decoder_body,
        grid=(B,),
        in_specs=[
            pl.BlockSpec((1, 128, _K), lambda b: (b, 0, 0)),
            pl.BlockSpec((1, _K, _DIM), lambda b: (b, 0, 0)),
            pl.BlockSpec((1, 1, _DIM), lambda b: (b, 0, 0)),
            full((128, 128)), full((128, _DIM)), full((128, _DIM)),
            full((128, 128)), full((128, _DIM)), full((128, _DIM)),
            full((128, 1)), full((128, 1)), full((128, 1)),
            full((128, 128)), full((128, 1)),
            full((64, 128)), full((64, 1)), full((3, 64)), full((1, 3)),
        ],
        out_specs=pl.BlockSpec((1, _K, 3), lambda b: (b, 0, 0)),
        out_shape=jax.ShapeDtypeStruct((B, _K, 3), jnp.float32),
        compiler_params=cp,
    )(x1, sampled, global_feat.reshape(B, 1, _DIM),
      a1, b1, c1, sa, sb, sc,
      ps1_b.reshape(128, 1), m1c1_b.reshape(128, 1), m1sc_b.reshape(128, 1),
      m1c2_w, m1c2_b.reshape(128, 1), m2c1_w, m2c1_b.reshape(64, 1),
      m2c2_w, m2c2_b.reshape(1, 3))
    return out


# speedup vs baseline: 2.2108x; 1.0478x over previous
import jax
import jax.numpy as jnp
from jax.experimental import pallas as pl
from jax.experimental.pallas import tpu as pltpu

_DIM = 1024
_N = 2048
_K = 64


def _fps_body(pts_ref, out_ref):
    iota = jax.lax.broadcasted_iota(jnp.int32, (_N, 1), 0)
    p = pts_ref[0]
    pn = jnp.sum(p * p, axis=1, keepdims=True)

    def step(i, carry):
        dist, far = carry
        c_row = pts_ref[0, pl.ds(far, 1), :]
        out_ref[0, pl.ds(i, 1), :] = c_row
        s = jax.lax.dot_general(p, c_row, (((1,), (1,)), ((), ())),
                                preferred_element_type=jnp.float32)
        cn = jnp.sum(c_row * c_row)
        d = pn - 2.0 * s + cn
        dist = jnp.minimum(dist, d)
        m = jnp.max(dist)
        nxt = jnp.min(jnp.where(dist == m, iota, _N))
        return dist, nxt

    dist0 = jnp.full((_N, 1), 1e10, dtype=jnp.float32)
    jax.lax.fori_loop(0, _K, step, (dist0, jnp.int32(0)))


def _ps1_body(g_ref, m_ref, out_ref):
    out_ref[...] = jnp.dot(g_ref[...], m_ref[...],
                           preferred_element_type=jnp.float32)


def _decoder_body(x1_ref, f_ref, g_ref,
                  a1_ref, b1_ref, c1_ref, sa_ref, sb_ref, sc_ref,
                  ps1b_ref, m1c1b_ref, m1scb_ref, m1c2w_ref, m1c2b_ref,
                  m2c1w_ref, m2c1b_ref, m2c2w_ref, m2c2b_ref, out_ref):
    dn = (((1,), (1,)), ((), ()))

    x1 = x1_ref[0] + ps1b_ref[...]
    f = f_ref[0]
    g = g_ref[0]

    def mm(w, x):
        return jnp.dot(w, x, preferred_element_type=jnp.float32)

    def mmt(w, x):
        return jax.lax.dot_general(w, x, dn, preferred_element_type=jnp.float32)

    h1 = mm(a1_ref[...], x1) + mmt(b1_ref[...], f) + mmt(c1_ref[...], g) \
        + m1c1b_ref[...]
    shortcut = mm(sa_ref[...], x1) + mmt(sb_ref[...], f) + mmt(sc_ref[...], g) \
        + m1scb_ref[...]
    h = mm(m1c2w_ref[...], jax.nn.relu(h1)) + m1c2b_ref[...] + shortcut
    r2 = jax.nn.relu(mm(m2c1w_ref[...], h) + m2c1b_ref[...])
    out_ref[0] = jax.lax.dot_general(
        r2, m2c2w_ref[...], (((0,), (1,)), ((), ())),
        preferred_element_type=jnp.float32) + m2c2b_ref[...]


def kernel(point_feat, global_feat, ps1_w, ps1_b, m1c1_w, m1c1_b, m1c2_w,
           m1c2_b, m1sc_w, m1sc_b, m2c1_w, m2c1_b, m2c2_w, m2c2_b):
    B = point_feat.shape[0]
    pts = jnp.transpose(point_feat, (0, 2, 1))

    cp = pltpu.CompilerParams(
        dimension_semantics=("arbitrary",),
        vmem_limit_bytes=100 * 1024 * 1024,
    )

    sampled = pl.pallas_call(
        _fps_body,
        grid=(B,),
        in_specs=[pl.BlockSpec((1, _N, _DIM), lambda b: (b, 0, 0))],
        out_specs=pl.BlockSpec((1, _K, _DIM), lambda b: (b, 0, 0)),
        out_shape=jax.ShapeDtypeStruct((B, _K, _DIM), jnp.float32),
        compiler_params=cp,
    )(pts)

    m = ps1_w.reshape(_DIM, 128 * _K)
    chunk = 1024
    x1_flat = pl.pallas_call(
        _ps1_body,
        grid=(128 * _K // chunk,),
        in_specs=[
            pl.BlockSpec((B, _DIM), lambda j: (0, 0)),
            pl.BlockSpec((_DIM, chunk), lambda j: (0, j)),
        ],
        out_specs=pl.BlockSpec((B, chunk), lambda j: (0, j)),
        out_shape=jax.ShapeDtypeStruct((B, 128 * _K), jnp.float32),
        compiler_params=cp,
    )(global_feat, m)
    x1 = x1_flat.reshape(B, 128, _K)

    a1, b1, c1 = m1c1_w[:, :128], m1c1_w[:, 128:128 + _DIM], m1c1_w[:, 128 + _DIM:]
    sa, sb, sc = m1sc_w[:, :128], m1sc_w[:, 128:128 + _DIM], m1sc_w[:, 128 + _DIM:]

    full = lambda shape: pl.BlockSpec(shape, lambda b: tuple(0 for _ in shape))
    out = pl.pallas_call(
        _decoder_body,
        grid=(B,),
        in_specs=[
            pl.BlockSpec((1, 128, _K), lambda b: (b, 0, 0)),
            pl.BlockSpec((1, _K, _DIM), lambda b: (b, 0, 0)),
            pl.BlockSpec((1, 1, _DIM), lambda b: (b, 0, 0)),
            full((128, 128)), full((128, _DIM)), full((128, _DIM)),
            full((128, 128)), full((128, _DIM)), full((128, _DIM)),
            full((128, 1)), full((128, 1)), full((128, 1)),
            full((128, 128)), full((128, 1)),
            full((64, 128)), full((64, 1)), full((3, 64)), full((1, 3)),
        ],
        out_specs=pl.BlockSpec((1, _K, 3), lambda b: (b, 0, 0)),
        out_shape=jax.ShapeDtypeStruct((B, _K, 3), jnp.float32),
        compiler_params=cp,
    )(x1, sampled, global_feat.reshape(B, 1, _DIM),
      a1, b1, c1, sa, sb, sc,
      ps1_b.reshape(128, 1), m1c1_b.reshape(128, 1), m1sc_b.reshape(128, 1),
      m1c2_w, m1c2_b.reshape(128, 1), m2c1_w, m2c1_b.reshape(64, 1),
      m2c2_w, m2c2_b.reshape(1, 3))
    return out


# speedup vs baseline: 2.2193x; 1.0039x over previous
import jax
import jax.numpy as jnp
from jax.experimental import pallas as pl
from jax.experimental.pallas import tpu as pltpu

_DIM = 1024
_N = 2048
_K = 64


def _fps_body(pts_ref, out_ref):
    iota = jax.lax.broadcasted_iota(jnp.int32, (_N, 1), 0)
    pn0 = jnp.sum(pts_ref[0] * pts_ref[0], axis=1, keepdims=True)

    def step(i, carry):
        dist, pn, far = carry
        c_row = pts_ref[0, pl.ds(far, 1), :]
        out_ref[0, pl.ds(i, 1), :] = c_row
        p = pts_ref[0]
        s = jnp.sum(p * c_row, axis=1, keepdims=True)
        cn = jnp.sum(c_row * c_row)
        d = pn - 2.0 * s + cn
        dist = jnp.minimum(dist, d)
        m = jnp.max(dist)
        nxt = jnp.min(jnp.where(dist == m, iota, _N))
        return dist, pn, nxt

    dist0 = jnp.full((_N, 1), 1e10, dtype=jnp.float32)
    jax.lax.fori_loop(0, _K, step, (dist0, pn0, jnp.int32(0)))


def _ps1_body(g_ref, m_ref, out_ref):
    out_ref[...] = jnp.dot(g_ref[...], m_ref[...],
                           preferred_element_type=jnp.float32)


def _decoder_body(x1_ref, f_ref, g_ref,
                  a1_ref, b1_ref, c1_ref, sa_ref, sb_ref, sc_ref,
                  ps1b_ref, m1c1b_ref, m1scb_ref, m1c2w_ref, m1c2b_ref,
                  m2c1w_ref, m2c1b_ref, m2c2w_ref, m2c2b_ref, out_ref):
    dn = (((1,), (1,)), ((), ()))

    x1 = x1_ref[0] + ps1b_ref[...]
    f = f_ref[0]
    g = g_ref[0]

    def mm(w, x):
        return jnp.dot(w, x, preferred_element_type=jnp.float32)

    def mmt(w, x):
        return jax.lax.dot_general(w, x, dn, preferred_element_type=jnp.float32)

    h1 = mm(a1_ref[...], x1) + mmt(b1_ref[...], f) + mmt(c1_ref[...], g) \
        + m1c1b_ref[...]
    shortcut = mm(sa_ref[...], x1) + mmt(sb_ref[...], f) + mmt(sc_ref[...], g) \
        + m1scb_ref[...]
    h = mm(m1c2w_ref[...], jax.nn.relu(h1)) + m1c2b_ref[...] + shortcut
    r2 = jax.nn.relu(mm(m2c1w_ref[...], h) + m2c1b_ref[...])
    out_ref[0] = jax.lax.dot_general(
        r2, m2c2w_ref[...], (((0,), (1,)), ((), ())),
        preferred_element_type=jnp.float32) + m2c2b_ref[...]


def kernel(point_feat, global_feat, ps1_w, ps1_b, m1c1_w, m1c1_b, m1c2_w,
           m1c2_b, m1sc_w, m1sc_b, m2c1_w, m2c1_b, m2c2_w, m2c2_b):
    B = point_feat.shape[0]
    pts = jnp.transpose(point_feat, (0, 2, 1))

    cp = pltpu.CompilerParams(
        dimension_semantics=("arbitrary",),
        vmem_limit_bytes=100 * 1024 * 1024,
    )

    sampled = pl.pallas_call(
        _fps_body,
        grid=(B,),
        in_specs=[pl.BlockSpec((1, _N, _DIM), lambda b: (b, 0, 0))],
        out_specs=pl.BlockSpec((1, _K, _DIM), lambda b: (b, 0, 0)),
        out_shape=jax.ShapeDtypeStruct((B, _K, _DIM), jnp.float32),
        compiler_params=cp,
    )(pts)

    m = ps1_w.reshape(_DIM, 128 * _K)
    chunk = 1024
    x1_flat = pl.pallas_call(
        _ps1_body,
        grid=(128 * _K // chunk,),
        in_specs=[
            pl.BlockSpec((B, _DIM), lambda j: (0, 0)),
            pl.BlockSpec((_DIM, chunk), lambda j: (0, j)),
        ],
        out_specs=pl.BlockSpec((B, chunk), lambda j: (0, j)),
        out_shape=jax.ShapeDtypeStruct((B, 128 * _K), jnp.float32),
        compiler_params=cp,
    )(global_feat, m)
    x1 = x1_flat.reshape(B, 128, _K)

    a1, b1, c1 = m1c1_w[:, :128], m1c1_w[:, 128:128 + _DIM], m1c1_w[:, 128 + _DIM:]
    sa, sb, sc = m1sc_w[:, :128], m1sc_w[:, 128:128 + _DIM], m1sc_w[:, 128 + _DIM:]

    full = lambda shape: pl.BlockSpec(shape, lambda b: tuple(0 for _ in shape))
    out = pl.pallas_call(
        _decoder_body,
        grid=(B,),
        in_specs=[
            pl.BlockSpec((1, 128, _K), lambda b: (b, 0, 0)),
            pl.BlockSpec((1, _K, _DIM), lambda b: (b, 0, 0)),
            pl.BlockSpec((1, 1, _DIM), lambda b: (b, 0, 0)),
            full((128, 128)), full((128, _DIM)), full((128, _DIM)),
            full((128, 128)), full((128, _DIM)), full((128, _DIM)),
            full((128, 1)), full((128, 1)), full((128, 1)),
            full((128, 128)), full((128, 1)),
            full((64, 128)), full((64, 1)), full((3, 64)), full((1, 3)),
        ],
        out_specs=pl.BlockSpec((1, _K, 3), lambda b: (b, 0, 0)),
        out_shape=jax.ShapeDtypeStruct((B, _K, 3), jnp.float32),
        compiler_params=cp,
    )(x1, sampled, global_feat.reshape(B, 1, _DIM),
      a1, b1, c1, sa, sb, sc,
      ps1_b.reshape(128, 1), m1c1_b.reshape(128, 1), m1sc_b.reshape(128, 1),
      m1c2_w, m1c2_b.reshape(128, 1), m2c1_w, m2c1_b.reshape(64, 1),
      m2c2_w, m2c2_b.reshape(1, 3))
    return out


# speedup vs baseline: 4.0449x; 1.8226x over previous
import jax
import jax.numpy as jnp
from jax.experimental import pallas as pl
from jax.experimental.pallas import tpu as pltpu

_DIM = 1024
_N = 2048
_K = 64


def _fps_body(pts_ref, out_ref, g_ref, pn_ref):
    p = pts_ref[0]
    pn = jnp.sum(p * p, axis=0, keepdims=True)
    g_ref[...] = jax.lax.dot_general(p, p, (((0,), (0,)), ((), ())),
                                     preferred_element_type=jnp.float32)
    pn_ref[...] = pn.T

    iota = jax.lax.broadcasted_iota(jnp.int32, (1, _N), 1)
    kiota = jax.lax.broadcasted_iota(jnp.int32, (1, _K), 1)

    def step(i, carry):
        dist, idxs, far = carry
        idxs = jnp.where(kiota == i, far, idxs)
        grow = g_ref[pl.ds(far, 1), :]
        cn = pn_ref[pl.ds(far, 1), 0]
        d = pn - 2.0 * grow + cn[0]
        dist = jnp.minimum(dist, d)
        m = jnp.max(dist)
        nxt = jnp.min(jnp.where(dist == m, iota, _N))
        return dist, idxs, nxt

    dist0 = jnp.full((1, _N), 1e10, dtype=jnp.float32)
    idxs0 = jnp.zeros((1, _K), dtype=jnp.int32)
    _, idxs, _ = jax.lax.fori_loop(0, _K, step, (dist0, idxs0, jnp.int32(0)))

    niota = jax.lax.broadcasted_iota(jnp.int32, (_N, _K), 0)
    onehot = jnp.where(niota == idxs, 1.0, 0.0).astype(jnp.float32)
    out_ref[0] = jnp.dot(p, onehot, preferred_element_type=jnp.float32)


def _ps1_body(g_ref, m_ref, out_ref):
    out_ref[...] = jnp.dot(g_ref[...], m_ref[...],
                           preferred_element_type=jnp.float32)


def _decoder_body(x1_ref, f_ref, g_ref,
                  a1_ref, b1_ref, c1_ref, sa_ref, sb_ref, sc_ref,
                  ps1b_ref, m1c1b_ref, m1scb_ref, m1c2w_ref, m1c2b_ref,
                  m2c1w_ref, m2c1b_ref, m2c2w_ref, m2c2b_ref, out_ref):
    dn = (((1,), (1,)), ((), ()))

    x1 = x1_ref[0] + ps1b_ref[...]
    f = f_ref[0]
    g = g_ref[0]

    def mm(w, x):
        return jnp.dot(w, x, preferred_element_type=jnp.float32)

    def mmt(w, x):
        return jax.lax.dot_general(w, x, dn, preferred_element_type=jnp.float32)

    h1 = mm(a1_ref[...], x1) + mm(b1_ref[...], f) + mmt(c1_ref[...], g) \
        + m1c1b_ref[...]
    shortcut = mm(sa_ref[...], x1) + mm(sb_ref[...], f) + mmt(sc_ref[...], g) \
        + m1scb_ref[...]
    h = mm(m1c2w_ref[...], jax.nn.relu(h1)) + m1c2b_ref[...] + shortcut
    r2 = jax.nn.relu(mm(m2c1w_ref[...], h) + m2c1b_ref[...])
    out_ref[0] = jax.lax.dot_general(
        r2, m2c2w_ref[...], (((0,), (1,)), ((), ())),
        preferred_element_type=jnp.float32) + m2c2b_ref[...]


def kernel(point_feat, global_feat, ps1_w, ps1_b, m1c1_w, m1c1_b, m1c2_w,
           m1c2_b, m1sc_w, m1sc_b, m2c1_w, m2c1_b, m2c2_w, m2c2_b):
    B = point_feat.shape[0]

    cp = pltpu.CompilerParams(
        dimension_semantics=("arbitrary",),
        vmem_limit_bytes=100 * 1024 * 1024,
    )

    sampled = pl.pallas_call(
        _fps_body,
        grid=(B,),
        in_specs=[pl.BlockSpec((1, _DIM, _N), lambda b: (b, 0, 0))],
        out_specs=pl.BlockSpec((1, _DIM, _K), lambda b: (b, 0, 0)),
        out_shape=jax.ShapeDtypeStruct((B, _DIM, _K), jnp.float32),
        scratch_shapes=[
            pltpu.VMEM((_N, _N), jnp.float32),
            pltpu.VMEM((_N, 1), jnp.float32),
        ],
        compiler_params=cp,
    )(point_feat)

    m = ps1_w.reshape(_DIM, 128 * _K)
    chunk = 1024
    x1_flat = pl.pallas_call(
        _ps1_body,
        grid=(128 * _K // chunk,),
        in_specs=[
            pl.BlockSpec((B, _DIM), lambda j: (0, 0)),
            pl.BlockSpec((_DIM, chunk), lambda j: (0, j)),
        ],
        out_specs=pl.BlockSpec((B, chunk), lambda j: (0, j)),
        out_shape=jax.ShapeDtypeStruct((B, 128 * _K), jnp.float32),
        compiler_params=cp,
    )(global_feat, m)
    x1 = x1_flat.reshape(B, 128, _K)

    a1, b1, c1 = m1c1_w[:, :128], m1c1_w[:, 128:128 + _DIM], m1c1_w[:, 128 + _DIM:]
    sa, sb, sc = m1sc_w[:, :128], m1sc_w[:, 128:128 + _DIM], m1sc_w[:, 128 + _DIM:]

    full = lambda shape: pl.BlockSpec(shape, lambda b: tuple(0 for _ in shape))
    out = pl.pallas_call(
        _decoder_body,
        grid=(B,),
        in_specs=[
            pl.BlockSpec((1, 128, _K), lambda b: (b, 0, 0)),
            pl.BlockSpec((1, _DIM, _K), lambda b: (b, 0, 0)),
            pl.BlockSpec((1, 1, _DIM), lambda b: (b, 0, 0)),
            full((128, 128)), full((128, _DIM)), full((128, _DIM)),
            full((128, 128)), full((128, _DIM)), full((128, _DIM)),
            full((128, 1)), full((128, 1)), full((128, 1)),
            full((128, 128)), full((128, 1)),
            full((64, 128)), full((64, 1)), full((3, 64)), full((1, 3)),
        ],
        out_specs=pl.BlockSpec((1, _K, 3), lambda b: (b, 0, 0)),
        out_shape=jax.ShapeDtypeStruct((B, _K, 3), jnp.float32),
        compiler_params=cp,
    )(x1, sampled, global_feat.reshape(B, 1, _DIM),
      a1, b1, c1, sa, sb, sc,
      ps1_b.reshape(128, 1), m1c1_b.reshape(128, 1), m1sc_b.reshape(128, 1),
      m1c2_w, m1c2_b.reshape(128, 1), m2c1_w, m2c1_b.reshape(64, 1),
      m2c2_w, m2c2_b.reshape(1, 3))
    return out
